# baseline (speedup 1.0000x reference)
import functools

import jax
import jax.numpy as jnp
from jax import lax
from jax.experimental import pallas as pl
from jax.experimental.pallas import tpu as pltpu
from jax.experimental.pallas import tpu_sc as plsc

V = 1000
D = 128
B = 16384
K = 5
VP = 1024
NW = 32
BPW = B // NW
OPW = BPW * (K + 1)
CHUNK = 128
NCH = OPW // CHUNK
LANES = 16


def _ce_table_body(te_ref, ce_ref, bias_ref, out_ref):
    a = lax.dot_general(te_ref[...].astype(jnp.bfloat16),
                        ce_ref[...].astype(jnp.bfloat16),
                        (((1,), (1,)), ((), ())),
                        preferred_element_type=jnp.float32)
    x = a + bias_ref[0:1, :]
    relu = jnp.maximum(x, 0.0)
    s = jnp.log1p(jnp.exp(-jnp.abs(x)))
    pos = relu - x + s
    neg = relu + s
    pu = lax.bitcast_convert_type(pos.astype(jnp.bfloat16), jnp.uint16)
    nu = lax.bitcast_convert_type(neg.astype(jnp.bfloat16), jnp.uint16)
    packed = (pu.astype(jnp.uint32) | (nu.astype(jnp.uint32) << 16)).astype(jnp.int32)
    out_ref[...] = packed.reshape(TR * 8, 128)


TR = 128


def _build_ce_table(te, ce, bias_tile):
    return pl.pallas_call(
        _ce_table_body,
        grid=(VP // TR,),
        in_specs=[
            pl.BlockSpec((TR, D), lambda i: (i, 0)),
            pl.BlockSpec((VP, D), lambda i: (0, 0)),
            pl.BlockSpec((8, VP), lambda i: (0, 0)),
        ],
        out_specs=pl.BlockSpec((TR * 8, 128), lambda i: (i, 0)),
        out_shape=jax.ShapeDtypeStruct((VP * 8, 128), jnp.int32),
    )(te, ce, bias_tile)


def _gather_body(table_hbm, tgt_hbm, ctx_hbm, neg_hbm, out_hbm,
                 tgt_v, ctx_v, neg_v, idx_v, vals_v, out_v, in_sem, sems):
    wid = lax.axis_index("s") * 2 + lax.axis_index("c")
    base = wid * BPW
    in_cp = [
        pltpu.async_copy(tgt_hbm.at[pl.ds(base, BPW)], tgt_v, in_sem),
        pltpu.async_copy(ctx_hbm.at[pl.ds(base, BPW)], ctx_v, in_sem),
        pltpu.async_copy(neg_hbm.at[pl.ds(base * K, BPW * K)], neg_v, in_sem),
    ]
    for cp in in_cp:
        cp.wait()

    iota = lax.iota(jnp.int32, LANES)
    G = BPW // LANES
    NSC = BPW // CHUNK

    copies = []
    for k in range(K + 1):
        def build(g, carry, k=k):
            off = g * LANES
            t = tgt_v[pl.ds(off, LANES)]
            if k == 0:
                c = ctx_v[pl.ds(off, LANES)]
            else:
                c = plsc.load_gather(neg_v, [(off + iota) * K + (k - 1)])
            idx_v[pl.ds(k * BPW + off, LANES)] = lax.shift_left(t, 10) | c
            return carry

        lax.fori_loop(0, G, build, 0)
        for ch in range(NSC):
            s = k * BPW + ch * CHUNK
            copies.append(pltpu.async_copy(
                table_hbm.at[idx_v.at[pl.ds(s, CHUNK)]],
                vals_v.at[pl.ds(s, CHUNK)],
                sems[k]))

    for k in range(K + 1):
        for ch in range(NSC):
            copies[k * NSC + ch].wait()

        kvec = jnp.full((LANES,), k, jnp.int32)

        def unpack(g, carry, k=k, kvec=kvec):
            off = g * LANES
            v = vals_v[pl.ds(k * BPW + off, LANES)]
            if k == 0:
                f = plsc.bitcast(lax.shift_left(v, 16), jnp.float32)
            else:
                f = plsc.bitcast(v & jnp.int32(-65536), jnp.float32)
            plsc.store_scatter(out_v, [off + iota, kvec], f)
            return carry

        lax.fori_loop(0, G, unpack, 0)

    pltpu.sync_copy(out_v, out_hbm.at[pl.ds(base, BPW)])


@functools.cache
def _gather_call():
    return pl.kernel(
        _gather_body,
        out_type=jax.ShapeDtypeStruct((B, K + 1), jnp.float32),
        mesh=plsc.VectorSubcoreMesh(core_axis_name="c", subcore_axis_name="s"),
        compiler_params=pltpu.CompilerParams(needs_layout_passes=False),
        scratch_types=[
            pltpu.VMEM((BPW,), jnp.int32),
            pltpu.VMEM((BPW,), jnp.int32),
            pltpu.VMEM((BPW * K,), jnp.int32),
            pltpu.VMEM((OPW,), jnp.int32),
            pltpu.VMEM((OPW,), jnp.int32),
            pltpu.VMEM((BPW, K + 1), jnp.float32),
            pltpu.SemaphoreType.DMA,
            [pltpu.SemaphoreType.DMA] * (K + 1),
        ],
    )


def kernel(input_targets, input_contexts, target_embedding, context_embedding,
           biases, negative_samples):
    bias_tile = jnp.zeros((8, VP), jnp.float32).at[:, :V].set(biases)
    table = _build_ce_table(target_embedding, context_embedding, bias_tile)
    return _gather_call()(table.reshape(VP * VP),
                          input_targets, input_contexts, negative_samples)


# speedup vs baseline: 1.0324x; 1.0324x over previous
import functools

import jax
import jax.numpy as jnp
from jax import lax
from jax.experimental import pallas as pl
from jax.experimental.pallas import tpu as pltpu
from jax.experimental.pallas import tpu_sc as plsc

V = 1000
D = 128
B = 16384
K = 5
VP = 1024
NW = 32
BPW = B // NW
OPW = BPW * (K + 1)
CHUNK = 128
NCH = OPW // CHUNK
LANES = 16


def _ce_table_body(te_ref, ce_ref, bias_ref, out_ref):
    a = lax.dot_general(te_ref[...].astype(jnp.bfloat16),
                        ce_ref[...].astype(jnp.bfloat16),
                        (((1,), (1,)), ((), ())),
                        preferred_element_type=jnp.float32)
    x = a + bias_ref[0:1, :]
    relu = jnp.maximum(x, 0.0)
    s = jnp.log1p(jnp.exp(-jnp.abs(x)))
    pos = relu - x + s
    neg = relu + s
    pu = lax.bitcast_convert_type(pos.astype(jnp.bfloat16), jnp.uint16)
    nu = lax.bitcast_convert_type(neg.astype(jnp.bfloat16), jnp.uint16)
    packed = (pu.astype(jnp.uint32) | (nu.astype(jnp.uint32) << 16)).astype(jnp.int32)
    out_ref[...] = packed.reshape(TR * 8, 128)


TR = 128


def _build_ce_table(te, ce, bias_tile):
    return pl.pallas_call(
        _ce_table_body,
        grid=(VP // TR,),
        in_specs=[
            pl.BlockSpec((TR, D), lambda i: (i, 0)),
            pl.BlockSpec((VP, D), lambda i: (0, 0)),
            pl.BlockSpec((8, VP), lambda i: (0, 0)),
        ],
        out_specs=pl.BlockSpec((TR * 8, 128), lambda i: (i, 0)),
        out_shape=jax.ShapeDtypeStruct((VP * 8, 128), jnp.int32),
    )(te, ce, bias_tile)


def _gather_body(table_hbm, tgt_hbm, ctx_hbm, neg_hbm, out_hbm,
                 tgt_v, ctx_v, neg_v, idx_v, vals_v, out_v, in_sem, sems):
    wid = lax.axis_index("s") * 2 + lax.axis_index("c")
    base = wid * BPW
    in_cp = [
        pltpu.async_copy(tgt_hbm.at[pl.ds(base, BPW)], tgt_v, in_sem),
        pltpu.async_copy(ctx_hbm.at[pl.ds(base, BPW)], ctx_v, in_sem),
        pltpu.async_copy(neg_hbm.at[pl.ds(base * K, BPW * K)], neg_v, in_sem),
    ]
    for cp in in_cp:
        cp.wait()

    iota = lax.iota(jnp.int32, LANES)
    G = BPW // LANES
    NSC = BPW // CHUNK

    copies = []
    for k in range(K + 1):
        def build(g, carry, k=k):
            off = g * LANES
            t = tgt_v[pl.ds(off, LANES)]
            if k == 0:
                c = ctx_v[pl.ds(off, LANES)]
            else:
                c = plsc.load_gather(neg_v, [(off + iota) * K + (k - 1)])
            idx_v[pl.ds(k * BPW + off, LANES)] = lax.shift_left(t, 10) | c
            return carry

        lax.fori_loop(0, G, build, 0)
        for ch in range(NSC):
            s = k * BPW + ch * CHUNK
            copies.append(pltpu.async_copy(
                table_hbm.at[idx_v.at[pl.ds(s, CHUNK)]],
                vals_v.at[pl.ds(s, CHUNK)],
                sems[k]))

    for k in range(K + 1):
        for ch in range(NSC):
            copies[k * NSC + ch].wait()

        kvec = jnp.full((LANES,), k, jnp.int32)

        def unpack(g, carry, k=k, kvec=kvec):
            off = g * LANES
            v = vals_v[pl.ds(k * BPW + off, LANES)]
            if k == 0:
                f = plsc.bitcast(lax.shift_left(v, 16), jnp.float32)
            else:
                f = plsc.bitcast(v & jnp.int32(-65536), jnp.float32)
            plsc.store_scatter(out_v, [off + iota, kvec], f)
            return carry

        lax.fori_loop(0, G, unpack, 0)

    pltpu.sync_copy(out_v, out_hbm.at[pl.ds(base, BPW)])


@functools.cache
def _gather_call():
    return pl.kernel(
        _gather_body,
        out_type=jax.ShapeDtypeStruct((B, K + 1), jnp.float32),
        mesh=plsc.VectorSubcoreMesh(core_axis_name="c", subcore_axis_name="s"),
        compiler_params=pltpu.CompilerParams(needs_layout_passes=False,
                                             skip_device_barrier=True),
        scratch_types=[
            pltpu.VMEM((BPW,), jnp.int32),
            pltpu.VMEM((BPW,), jnp.int32),
            pltpu.VMEM((BPW * K,), jnp.int32),
            pltpu.VMEM((OPW,), jnp.int32),
            pltpu.VMEM((OPW,), jnp.int32),
            pltpu.VMEM((BPW, K + 1), jnp.float32),
            pltpu.SemaphoreType.DMA,
            [pltpu.SemaphoreType.DMA] * (K + 1),
        ],
    )


def kernel(input_targets, input_contexts, target_embedding, context_embedding,
           biases, negative_samples):
    bias_p = jnp.zeros((VP,), jnp.float32).at[:V].set(biases)
    bias_tile = jnp.tile(bias_p[None, :], (8, 1))
    table = _build_ce_table(target_embedding, context_embedding, bias_tile)
    return _gather_call()(table.reshape(VP * VP),
                          input_targets, input_contexts, negative_samples)


# speedup vs baseline: 1.0644x; 1.0309x over previous
import functools

import jax
import jax.numpy as jnp
from jax import lax
from jax.experimental import pallas as pl
from jax.experimental.pallas import tpu as pltpu
from jax.experimental.pallas import tpu_sc as plsc

V = 1000
D = 128
B = 16384
K = 5
VP = 1024
NW = 32
BPW = B // NW
OPW = BPW * (K + 1)
CHUNK = 128
NCH = OPW // CHUNK
LANES = 16


def _ce_table_body(te_ref, ce_ref, bias_ref, out_ref):
    a = lax.dot_general(te_ref[...].astype(jnp.bfloat16),
                        ce_ref[...].astype(jnp.bfloat16),
                        (((1,), (1,)), ((), ())),
                        preferred_element_type=jnp.float32)
    x = a + bias_ref[0:1, :]
    relu = jnp.maximum(x, 0.0)
    s = jnp.log1p(jnp.exp(-jnp.abs(x)))
    pos = relu - x + s
    neg = relu + s
    pu = lax.bitcast_convert_type(pos.astype(jnp.bfloat16), jnp.uint16)
    nu = lax.bitcast_convert_type(neg.astype(jnp.bfloat16), jnp.uint16)
    packed = (pu.astype(jnp.uint32) | (nu.astype(jnp.uint32) << 16)).astype(jnp.int32)
    out_ref[...] = packed.reshape(TR * 8, 128)


TR = 256


def _build_ce_table(te, ce, bias_tile):
    return pl.pallas_call(
        _ce_table_body,
        grid=(VP // TR,),
        in_specs=[
            pl.BlockSpec((TR, D), lambda i: (i, 0)),
            pl.BlockSpec((VP, D), lambda i: (0, 0)),
            pl.BlockSpec((8, VP), lambda i: (0, 0)),
        ],
        out_specs=pl.BlockSpec((TR * 8, 128), lambda i: (i, 0)),
        out_shape=jax.ShapeDtypeStruct((VP * 8, 128), jnp.int32),
    )(te, ce, bias_tile)


def _gather_body(table_hbm, tgt_hbm, ctx_hbm, neg_hbm, out_hbm,
                 tgt_v, ctx_v, neg_v, idx_v, vals_v, out_v, in_sem, sems):
    wid = lax.axis_index("s") * 2 + lax.axis_index("c")
    base = wid * BPW
    in_cp = [
        pltpu.async_copy(tgt_hbm.at[pl.ds(base, BPW)], tgt_v, in_sem),
        pltpu.async_copy(ctx_hbm.at[pl.ds(base, BPW)], ctx_v, in_sem),
        pltpu.async_copy(neg_hbm.at[pl.ds(base * K, BPW * K)], neg_v, in_sem),
    ]
    for cp in in_cp:
        cp.wait()

    iota = lax.iota(jnp.int32, LANES)
    G = BPW // LANES
    NSC = BPW // CHUNK

    copies = []
    for k in range(K + 1):
        def build(g, carry, k=k):
            off = g * LANES
            t = tgt_v[pl.ds(off, LANES)]
            if k == 0:
                c = ctx_v[pl.ds(off, LANES)]
            else:
                c = plsc.load_gather(neg_v, [(off + iota) * K + (k - 1)])
            idx_v[pl.ds(k * BPW + off, LANES)] = lax.shift_left(t, 10) | c
            return carry

        lax.fori_loop(0, G, build, 0)
        for ch in range(NSC):
            s = k * BPW + ch * CHUNK
            copies.append(pltpu.async_copy(
                table_hbm.at[idx_v.at[pl.ds(s, CHUNK)]],
                vals_v.at[pl.ds(s, CHUNK)],
                sems[k]))

    for k in range(K + 1):
        for ch in range(NSC):
            copies[k * NSC + ch].wait()

        kvec = jnp.full((LANES,), k, jnp.int32)

        def unpack(g, carry, k=k, kvec=kvec):
            off = g * LANES
            v = vals_v[pl.ds(k * BPW + off, LANES)]
            if k == 0:
                f = plsc.bitcast(lax.shift_left(v, 16), jnp.float32)
            else:
                f = plsc.bitcast(v & jnp.int32(-65536), jnp.float32)
            plsc.store_scatter(out_v, [off + iota, kvec], f)
            return carry

        lax.fori_loop(0, G, unpack, 0)

    pltpu.sync_copy(out_v, out_hbm.at[pl.ds(base, BPW)])


@functools.cache
def _gather_call():
    return pl.kernel(
        _gather_body,
        out_type=jax.ShapeDtypeStruct((B, K + 1), jnp.float32),
        mesh=plsc.VectorSubcoreMesh(core_axis_name="c", subcore_axis_name="s"),
        compiler_params=pltpu.CompilerParams(needs_layout_passes=False),
        scratch_types=[
            pltpu.VMEM((BPW,), jnp.int32),
            pltpu.VMEM((BPW,), jnp.int32),
            pltpu.VMEM((BPW * K,), jnp.int32),
            pltpu.VMEM((OPW,), jnp.int32),
            pltpu.VMEM((OPW,), jnp.int32),
            pltpu.VMEM((BPW, K + 1), jnp.float32),
            pltpu.SemaphoreType.DMA,
            [pltpu.SemaphoreType.DMA] * (K + 1),
        ],
    )


def kernel(input_targets, input_contexts, target_embedding, context_embedding,
           biases, negative_samples):
    bias_p = jnp.zeros((VP,), jnp.float32).at[:V].set(biases)
    bias_tile = jnp.tile(bias_p[None, :], (8, 1))
    table = _build_ce_table(target_embedding, context_embedding, bias_tile)
    return _gather_call()(table.reshape(VP * VP),
                          input_targets, input_contexts, negative_samples)


# speedup vs baseline: 1.0679x; 1.0033x over previous
import functools

import jax
import jax.numpy as jnp
from jax import lax
from jax.experimental import pallas as pl
from jax.experimental.pallas import tpu as pltpu
from jax.experimental.pallas import tpu_sc as plsc

V = 1000
D = 128
B = 16384
K = 5
VP = 1024
NW = 32
BPW = B // NW
OPW = BPW * (K + 1)
CHUNK = 128
NCH = OPW // CHUNK
LANES = 16


def _ce_table_body(te_ref, ce_ref, bias_ref, out_ref):
    a = lax.dot_general(te_ref[...].astype(jnp.bfloat16),
                        ce_ref[...].astype(jnp.bfloat16),
                        (((1,), (1,)), ((), ())),
                        preferred_element_type=jnp.float32)
    x = a + bias_ref[0:1, :]
    relu = jnp.maximum(x, 0.0)
    s = jnp.log1p(jnp.exp(-jnp.abs(x)))
    pos = relu - x + s
    neg = relu + s
    pu = lax.bitcast_convert_type(pos.astype(jnp.bfloat16), jnp.uint16)
    nu = lax.bitcast_convert_type(neg.astype(jnp.bfloat16), jnp.uint16)
    packed = (pu.astype(jnp.uint32) | (nu.astype(jnp.uint32) << 16)).astype(jnp.int32)
    out_ref[...] = packed.reshape(TR * 8, 128)


TR = 512


def _build_ce_table(te, ce, bias_tile):
    return pl.pallas_call(
        _ce_table_body,
        grid=(VP // TR,),
        in_specs=[
            pl.BlockSpec((TR, D), lambda i: (i, 0)),
            pl.BlockSpec((VP, D), lambda i: (0, 0)),
            pl.BlockSpec((8, VP), lambda i: (0, 0)),
        ],
        out_specs=pl.BlockSpec((TR * 8, 128), lambda i: (i, 0)),
        out_shape=jax.ShapeDtypeStruct((VP * 8, 128), jnp.int32),
    )(te, ce, bias_tile)


def _gather_body(table_hbm, tgt_hbm, ctx_hbm, neg_hbm, out_hbm,
                 tgt_v, ctx_v, neg_v, idx_v, vals_v, out_v, in_sem, sems):
    wid = lax.axis_index("s") * 2 + lax.axis_index("c")
    base = wid * BPW
    in_cp = [
        pltpu.async_copy(tgt_hbm.at[pl.ds(base, BPW)], tgt_v, in_sem),
        pltpu.async_copy(ctx_hbm.at[pl.ds(base, BPW)], ctx_v, in_sem),
        pltpu.async_copy(neg_hbm.at[pl.ds(base * K, BPW * K)], neg_v, in_sem),
    ]
    for cp in in_cp:
        cp.wait()

    iota = lax.iota(jnp.int32, LANES)
    G = BPW // LANES
    NSC = BPW // CHUNK

    copies = []
    for k in range(K + 1):
        def build(g, carry, k=k):
            off = g * LANES
            t = tgt_v[pl.ds(off, LANES)]
            if k == 0:
                c = ctx_v[pl.ds(off, LANES)]
            else:
                c = plsc.load_gather(neg_v, [(off + iota) * K + (k - 1)])
            idx_v[pl.ds(k * BPW + off, LANES)] = lax.shift_left(t, 10) | c
            return carry

        lax.fori_loop(0, G, build, 0)
        for ch in range(NSC):
            s = k * BPW + ch * CHUNK
            copies.append(pltpu.async_copy(
                table_hbm.at[idx_v.at[pl.ds(s, CHUNK)]],
                vals_v.at[pl.ds(s, CHUNK)],
                sems[k]))

    for k in range(K + 1):
        for ch in range(NSC):
            copies[k * NSC + ch].wait()

        kvec = jnp.full((LANES,), k, jnp.int32)

        def unpack(g, carry, k=k, kvec=kvec):
            off = g * LANES
            v = vals_v[pl.ds(k * BPW + off, LANES)]
            if k == 0:
                f = plsc.bitcast(lax.shift_left(v, 16), jnp.float32)
            else:
                f = plsc.bitcast(v & jnp.int32(-65536), jnp.float32)
            plsc.store_scatter(out_v, [off + iota, kvec], f)
            return carry

        lax.fori_loop(0, G, unpack, 0)

    pltpu.sync_copy(out_v, out_hbm.at[pl.ds(base, BPW)])


@functools.cache
def _gather_call():
    return pl.kernel(
        _gather_body,
        out_type=jax.ShapeDtypeStruct((B, K + 1), jnp.float32),
        mesh=plsc.VectorSubcoreMesh(core_axis_name="c", subcore_axis_name="s"),
        compiler_params=pltpu.CompilerParams(needs_layout_passes=False),
        scratch_types=[
            pltpu.VMEM((BPW,), jnp.int32),
            pltpu.VMEM((BPW,), jnp.int32),
            pltpu.VMEM((BPW * K,), jnp.int32),
            pltpu.VMEM((OPW,), jnp.int32),
            pltpu.VMEM((OPW,), jnp.int32),
            pltpu.VMEM((BPW, K + 1), jnp.float32),
            pltpu.SemaphoreType.DMA,
            [pltpu.SemaphoreType.DMA] * (K + 1),
        ],
    )


def kernel(input_targets, input_contexts, target_embedding, context_embedding,
           biases, negative_samples):
    bias_p = jnp.zeros((VP,), jnp.float32).at[:V].set(biases)
    bias_tile = jnp.tile(bias_p[None, :], (8, 1))
    table = _build_ce_table(target_embedding, context_embedding, bias_tile)
    return _gather_call()(table.reshape(VP * VP),
                          input_targets, input_contexts, negative_samples)


# speedup vs baseline: 1.1047x; 1.0344x over previous
import functools

import jax
import jax.numpy as jnp
from jax import lax
from jax.experimental import pallas as pl
from jax.experimental.pallas import tpu as pltpu
from jax.experimental.pallas import tpu_sc as plsc

V = 1000
D = 128
B = 16384
K = 5
VP = 1024
NW = 32
BPW = B // NW
OPW = BPW * (K + 1)
CHUNK = 128
NCH = OPW // CHUNK
LANES = 16


def _ce_table_body(te_ref, ce_ref, bias_ref, out_ref):
    a = lax.dot_general(te_ref[...].astype(jnp.bfloat16),
                        ce_ref[...].astype(jnp.bfloat16),
                        (((1,), (1,)), ((), ())),
                        preferred_element_type=jnp.float32)
    x = a + bias_ref[...].reshape(1, VP)
    relu = jnp.maximum(x, 0.0)
    s = jnp.log1p(jnp.exp(-jnp.abs(x)))
    pos = relu - x + s
    neg = relu + s
    pu = lax.bitcast_convert_type(pos.astype(jnp.bfloat16), jnp.uint16)
    nu = lax.bitcast_convert_type(neg.astype(jnp.bfloat16), jnp.uint16)
    packed = (pu.astype(jnp.uint32) | (nu.astype(jnp.uint32) << 16)).astype(jnp.int32)
    out_ref[...] = packed.reshape(TR * 8, 128)


TR = 512


def _build_ce_table(te, ce, bias_tile):
    return pl.pallas_call(
        _ce_table_body,
        grid=(VP // TR,),
        in_specs=[
            pl.BlockSpec((TR, D), lambda i: (i, 0)),
            pl.BlockSpec((VP, D), lambda i: (0, 0)),
            pl.BlockSpec((VP,), lambda i: (0,)),
        ],
        out_specs=pl.BlockSpec((TR * 8, 128), lambda i: (i, 0)),
        out_shape=jax.ShapeDtypeStruct((VP * 8, 128), jnp.int32),
    )(te, ce, bias_tile)


def _gather_body(table_hbm, tgt_hbm, ctx_hbm, neg_hbm, out_hbm,
                 tgt_v, ctx_v, neg_v, idx_v, vals_v, out_v, in_sem, sems):
    wid = lax.axis_index("s") * 2 + lax.axis_index("c")
    base = wid * BPW
    in_cp = [
        pltpu.async_copy(tgt_hbm.at[pl.ds(base, BPW)], tgt_v, in_sem),
        pltpu.async_copy(ctx_hbm.at[pl.ds(base, BPW)], ctx_v, in_sem),
        pltpu.async_copy(neg_hbm.at[pl.ds(base * K, BPW * K)], neg_v, in_sem),
    ]
    for cp in in_cp:
        cp.wait()

    iota = lax.iota(jnp.int32, LANES)
    G = BPW // LANES
    NSC = BPW // CHUNK

    copies = []
    for k in range(K + 1):
        def build(g, carry, k=k):
            off = g * LANES
            t = tgt_v[pl.ds(off, LANES)]
            if k == 0:
                c = ctx_v[pl.ds(off, LANES)]
            else:
                c = plsc.load_gather(neg_v, [(off + iota) * K + (k - 1)])
            idx_v[pl.ds(k * BPW + off, LANES)] = lax.shift_left(t, 10) | c
            return carry

        lax.fori_loop(0, G, build, 0)
        for ch in range(NSC):
            s = k * BPW + ch * CHUNK
            copies.append(pltpu.async_copy(
                table_hbm.at[idx_v.at[pl.ds(s, CHUNK)]],
                vals_v.at[pl.ds(s, CHUNK)],
                sems[k]))

    for k in range(K + 1):
        for ch in range(NSC):
            copies[k * NSC + ch].wait()

        kvec = jnp.full((LANES,), k, jnp.int32)

        def unpack(g, carry, k=k, kvec=kvec):
            off = g * LANES
            v = vals_v[pl.ds(k * BPW + off, LANES)]
            if k == 0:
                f = plsc.bitcast(lax.shift_left(v, 16), jnp.float32)
            else:
                f = plsc.bitcast(v & jnp.int32(-65536), jnp.float32)
            plsc.store_scatter(out_v, [off + iota, kvec], f)
            return carry

        lax.fori_loop(0, G, unpack, 0)

    pltpu.sync_copy(out_v, out_hbm.at[pl.ds(base, BPW)])


@functools.cache
def _gather_call():
    return pl.kernel(
        _gather_body,
        out_type=jax.ShapeDtypeStruct((B, K + 1), jnp.float32),
        mesh=plsc.VectorSubcoreMesh(core_axis_name="c", subcore_axis_name="s"),
        compiler_params=pltpu.CompilerParams(needs_layout_passes=False),
        scratch_types=[
            pltpu.VMEM((BPW,), jnp.int32),
            pltpu.VMEM((BPW,), jnp.int32),
            pltpu.VMEM((BPW * K,), jnp.int32),
            pltpu.VMEM((OPW,), jnp.int32),
            pltpu.VMEM((OPW,), jnp.int32),
            pltpu.VMEM((BPW, K + 1), jnp.float32),
            pltpu.SemaphoreType.DMA,
            [pltpu.SemaphoreType.DMA] * (K + 1),
        ],
    )


def kernel(input_targets, input_contexts, target_embedding, context_embedding,
           biases, negative_samples):
    table = _build_ce_table(target_embedding, context_embedding, biases)
    return _gather_call()(table.reshape(VP * VP),
                          input_targets, input_contexts, negative_samples)
